# baseline (speedup 1.0000x reference)
import functools

import jax
import jax.numpy as jnp
from jax import lax
from jax.experimental import pallas as pl
from jax.experimental.pallas import tpu as pltpu
from jax.experimental.pallas import tpu_sc as plsc

CHUNK = 4


@functools.lru_cache(maxsize=None)
def _build(B, L, EMB):
    info = plsc.get_sparse_core_info()
    NC, NS = info.num_cores, info.num_subcores
    NW = NC * NS
    RPW = B // NW
    NCH = RPW // CHUNK

    @functools.partial(
        pl.kernel,
        mesh=plsc.VectorSubcoreMesh(core_axis_name="c", subcore_axis_name="s"),
        compiler_params=pltpu.CompilerParams(use_tc_tiling_on_sc=False),
        out_type=jax.ShapeDtypeStruct((B, L, EMB), jnp.float32),
        scratch_types=[
            pltpu.VMEM((CHUNK, L), jnp.int32),
            pltpu.VMEM((CHUNK, L), jnp.int32),
            pltpu.VMEM((CHUNK, L, EMB), jnp.float32),
            pltpu.VMEM((CHUNK, L, EMB), jnp.float32),
            pltpu.VMEM((L, EMB), jnp.float32),
            pltpu.SemaphoreType.DMA,
            pltpu.SemaphoreType.DMA,
            pltpu.SemaphoreType.DMA,
        ],
    )
    def k(x_hbm, emb_hbm, pos_hbm, out_hbm, ib0, ib1, rb0, rb1, pos_v,
          gsem, osem, isem):
        wid = lax.axis_index("s") * NC + lax.axis_index("c")
        base = wid * RPW
        ibufs = (ib0, ib1)
        rbufs = (rb0, rb1)

        def idx_start(c, ib):
            pltpu.async_copy(x_hbm.at[pl.ds(base + c * CHUNK, CHUNK)], ib, isem)

        def idx_wait():
            pltpu.make_async_copy(
                x_hbm.at[pl.ds(base, CHUNK)], ib0, isem).wait()

        def gather_start(ib, rb):
            for r in range(CHUNK):
                pltpu.async_copy(emb_hbm.at[ib.at[r]], rb.at[r], gsem)

        def gather_wait():
            for r in range(CHUNK):
                pltpu.make_async_copy(
                    emb_hbm.at[ib0.at[r]], rb0.at[r], gsem).wait()

        def out_start(c, rb):
            pltpu.async_copy(
                rb, out_hbm.at[pl.ds(base + c * CHUNK, CHUNK)], osem)

        def out_wait():
            pltpu.make_async_copy(
                rb0, out_hbm.at[pl.ds(base, CHUNK)], osem).wait()

        def vadd(rb):
            def add_i(i, carry):
                for j in range(EMB // 16):
                    sl = pl.ds(j * 16, 16)
                    p = pos_v[i, sl]
                    for r in range(CHUNK):
                        rb[r, i, sl] = rb[r, i, sl] + p
                return carry
            lax.fori_loop(0, L, add_i, 0)

        pltpu.sync_copy(pos_hbm.at[pl.ds(0, L)], pos_v)
        pltpu.sync_copy(x_hbm.at[pl.ds(base, CHUNK)], ib0)
        gather_start(ib0, rb0)
        idx_start(1, ib1)

        for c in range(NCH):
            A = c & 1
            if c + 1 < NCH:
                idx_wait()
                if c >= 1:
                    out_wait()
                gather_start(ibufs[1 - A], rbufs[1 - A])
            elif c >= 1:
                out_wait()
            gather_wait()
            if c + 2 < NCH:
                idx_start(c + 2, ibufs[A])
            vadd(rbufs[A])
            out_start(c, rbufs[A])
        out_wait()

    return k


def kernel(x, emb_table, pos_table):
    B, L = x.shape
    EMB = emb_table.shape[1]
    k = _build(B // 2, L, EMB)
    xi = x.astype(jnp.int32)
    o1 = k(xi[: B // 2], emb_table, pos_table)
    o2 = k(xi[B // 2 :], emb_table, pos_table)
    return jnp.concatenate([o1, o2], axis=0)


# speedup vs baseline: 1.1906x; 1.1906x over previous
import functools

import jax
import jax.numpy as jnp
from jax import lax
from jax.experimental import pallas as pl
from jax.experimental.pallas import tpu as pltpu
from jax.experimental.pallas import tpu_sc as plsc

CHUNK = 4


@functools.lru_cache(maxsize=None)
def _build(B, L, EMB):
    info = plsc.get_sparse_core_info()
    NC, NS = info.num_cores, info.num_subcores
    NW = NC * NS
    RPW = B // NW
    NCH = RPW // CHUNK

    @functools.partial(
        pl.kernel,
        mesh=plsc.VectorSubcoreMesh(core_axis_name="c", subcore_axis_name="s"),
        compiler_params=pltpu.CompilerParams(use_tc_tiling_on_sc=False),
        out_type=jax.ShapeDtypeStruct((B, L, EMB), jnp.float32),
        scratch_types=[
            pltpu.VMEM((CHUNK, L), jnp.int32),
            pltpu.VMEM((CHUNK, L), jnp.int32),
            pltpu.VMEM((CHUNK, L, EMB), jnp.float32),
            pltpu.VMEM((CHUNK, L, EMB), jnp.float32),
            pltpu.VMEM((L, EMB), jnp.float32),
            pltpu.SemaphoreType.DMA,
            pltpu.SemaphoreType.DMA,
            pltpu.SemaphoreType.DMA,
        ],
    )
    def k(x_hbm, emb_hbm, pos_hbm, out_hbm, ib0, ib1, rb0, rb1, pos_v,
          gsem, osem, isem):
        wid = lax.axis_index("s") * NC + lax.axis_index("c")
        base = wid * RPW
        ibufs = (ib0, ib1)
        rbufs = (rb0, rb1)

        def idx_start(c, ib):
            pltpu.async_copy(x_hbm.at[pl.ds(base + c * CHUNK, CHUNK)], ib, isem)

        def idx_wait():
            pltpu.make_async_copy(
                x_hbm.at[pl.ds(base, CHUNK)], ib0, isem).wait()

        def gather_start(ib, rb):
            for r in range(CHUNK):
                pltpu.async_copy(emb_hbm.at[ib.at[r]], rb.at[r], gsem)

        def gather_wait():
            for r in range(CHUNK):
                pltpu.make_async_copy(
                    emb_hbm.at[ib0.at[r]], rb0.at[r], gsem).wait()

        def out_start(c, rb):
            pltpu.async_copy(
                rb, out_hbm.at[pl.ds(base + c * CHUNK, CHUNK)], osem)

        def out_wait():
            pltpu.make_async_copy(
                rb0, out_hbm.at[pl.ds(base, CHUNK)], osem).wait()

        def vadd(rb):
            def add_i(i, carry):
                for j in range(EMB // 16):
                    sl = pl.ds(j * 16, 16)
                    p = pos_v[i, sl]
                    for r in range(CHUNK):
                        rb[r, i, sl] = rb[r, i, sl] + p
                return carry
            lax.fori_loop(0, L, add_i, 0)

        pltpu.sync_copy(pos_hbm.at[pl.ds(0, L)], pos_v)
        pltpu.sync_copy(x_hbm.at[pl.ds(base, CHUNK)], ib0)
        gather_start(ib0, rb0)
        idx_start(1, ib1)

        for c in range(NCH):
            A = c & 1
            if c + 1 < NCH:
                idx_wait()
                if c >= 1:
                    out_wait()
                gather_start(ibufs[1 - A], rbufs[1 - A])
            elif c >= 1:
                out_wait()
            gather_wait()
            if c + 2 < NCH:
                idx_start(c + 2, ibufs[A])
            vadd(rbufs[A])
            out_start(c, rbufs[A])
        out_wait()

    return k


def kernel(x, emb_table, pos_table):
    B, L = x.shape
    EMB = emb_table.shape[1]
    k = _build(B, L, EMB)
    return k(x.astype(jnp.int32), emb_table, pos_table)
